# baseline (speedup 1.0000x reference)
import jax
import jax.numpy as jnp
from jax.experimental import pallas as pl

N = 10000
E = 320000
K = 64
EPS = 1e-05
NEG_SLOPE = 0.01


def _leaky(x):
    return jnp.where(x >= 0, x, NEG_SLOPE * x)


def _graph_conv(x, src, dst, ew, W):
    out_deg = jnp.clip(jnp.bincount(src, length=N), 1, None).astype(x.dtype)
    in_deg = jnp.clip(jnp.bincount(dst, length=N), 1, None).astype(x.dtype)
    h = x * (out_deg ** -0.5)[:, None]
    m = h[src] * ew[:, None]
    agg = jax.ops.segment_sum(m, dst, num_segments=N)
    agg = agg * (in_deg ** -0.5)[:, None]
    return agg @ W


def _graph_norm(x, alpha, gamma, beta):
    mean = jnp.mean(x, axis=0, keepdims=True)
    sub = x - alpha[None, :] * mean
    std = jnp.sqrt(jnp.mean(sub * sub, axis=0, keepdims=True) + EPS)
    return gamma[None, :] * sub / std + beta[None, :]


def _sort_pool(feat, k):
    fs = jnp.sort(feat, axis=-1)
    _, idx = jax.lax.top_k(fs[:, -1], k)
    sel = fs[idx]
    return sel.reshape(1, k * feat.shape[-1])


def _tail_kernel(x_ref, o_ref):
    x = x_ref[...]
    o_ref[...] = jnp.where(x >= 0, x, NEG_SLOPE * x)


def kernel(features, edge_index, edge_weights, W1, W2, alpha1, gamma1, beta1, alpha2, gamma2, beta2):
    src = edge_index[0]
    dst = edge_index[1]
    h1 = _graph_conv(features, src, dst, edge_weights, W1)
    h1 = _graph_norm(h1, alpha1, gamma1, beta1)
    h1 = _leaky(h1)
    r1 = _sort_pool(h1, K)
    h2 = _graph_conv(h1, src, dst, edge_weights, W2)
    h2 = _graph_norm(h2, alpha2, gamma2, beta2)
    h2 = _leaky(h2)
    r2 = _sort_pool(h2, K)
    readouts = jnp.concatenate([r1, r2], axis=1)
    return pl.pallas_call(
        _tail_kernel,
        out_shape=jax.ShapeDtypeStruct(readouts.shape, readouts.dtype),
    )(readouts)


# speedup vs baseline: 3.6492x; 3.6492x over previous
import functools

import jax
import jax.numpy as jnp
from jax import lax
from jax.experimental import pallas as pl
from jax.experimental.pallas import tpu as pltpu
from jax.experimental.pallas import tpu_sc as plsc

N = 10000
E = 320000
K = 64
EPS = 1e-05
NEG_SLOPE = 0.01

NC = 2
NS = 16
ET = E // NS
CHUNK = 80
NCHUNK = ET // CHUNK
RPW = 624
RCOPY = 208
RTAIL = N - NS * RPW


def _leaky(x):
    return jnp.where(x >= 0, x, NEG_SLOPE * x)


def _edge_scatter_body(D, y_hbm, src_hbm, dst_hbm, ew_hbm, out_hbm,
                       srcv, dstv, eww, idxb, dstb, rows, zbuf, acc, sem):
    cid = lax.axis_index("c")
    sid = lax.axis_index("s")

    def zero_row(r, _):
        for d in range(D // 16):
            zbuf[r, pl.ds(d * 16, 16)] = jnp.zeros((16,), jnp.float32)
        return 0

    lax.fori_loop(0, RCOPY, zero_row, 0)
    for j in range(RPW // RCOPY):
        pltpu.sync_copy(zbuf, acc.at[pl.ds(sid * RPW + j * RCOPY, RCOPY)])

    @pl.when(sid == NS - 1)
    def _():
        pltpu.sync_copy(zbuf.at[pl.ds(0, RTAIL)], acc.at[pl.ds(NS * RPW, RTAIL)])

    plsc.subcore_barrier()

    pltpu.sync_copy(src_hbm.at[sid], srcv)
    pltpu.sync_copy(dst_hbm.at[sid], dstv)
    pltpu.sync_copy(ew_hbm.at[pl.ds(sid * ET, ET)], eww)

    row_off = cid * N

    def chunk_body(i, _):
        def adj_idx(g, _):
            idxb[pl.ds(g * 16, 16)] = srcv[i, pl.ds(g * 16, 16)] + row_off
            dstb[pl.ds(g * 16, 16)] = dstv[i, pl.ds(g * 16, 16)]
            return 0

        lax.fori_loop(0, CHUNK // 16, adj_idx, 0)
        pltpu.async_copy(y_hbm.at[idxb], rows, sem).wait()

        def scale_group(g, _):
            ew16 = eww[pl.ds(i * CHUNK + g * 16, 16)]
            for j in range(16):
                w = jnp.broadcast_to(ew16[j], (16,))
                e = g * 16 + j
                for d in range(D // 16):
                    rows[e, pl.ds(d * 16, 16)] = rows[e, pl.ds(d * 16, 16)] * w
            return 0

        lax.fori_loop(0, CHUNK // 16, scale_group, 0)
        pltpu.sync_copy(rows, acc.at[dstb], add=True)
        return 0

    lax.fori_loop(0, NCHUNK, chunk_body, 0)
    plsc.subcore_barrier()

    for j in range(RPW // RCOPY):
        r0 = sid * RPW + j * RCOPY
        pltpu.sync_copy(acc.at[pl.ds(r0, RCOPY)],
                        out_hbm.at[pl.ds(cid * N + r0, RCOPY)])

    @pl.when(sid == NS - 1)
    def _():
        pltpu.sync_copy(acc.at[pl.ds(NS * RPW, RTAIL)],
                        out_hbm.at[pl.ds(cid * N + NS * RPW, RTAIL)])


def _sc_edge_scatter(y, src3, dst3, ew, D):
    ysplit = jnp.concatenate([y[:, :D], y[:, D:]], axis=0)
    mesh = plsc.VectorSubcoreMesh(core_axis_name="c", subcore_axis_name="s")
    body = functools.partial(_edge_scatter_body, D)
    f = pl.kernel(
        body,
        out_type=jax.ShapeDtypeStruct((NC * N, D), jnp.float32),
        mesh=mesh,
        compiler_params=pltpu.CompilerParams(use_tc_tiling_on_sc=False),
        scratch_types=[
            pltpu.VMEM((NCHUNK, CHUNK), jnp.int32),
            pltpu.VMEM((NCHUNK, CHUNK), jnp.int32),
            pltpu.VMEM((ET,), jnp.float32),
            pltpu.VMEM((CHUNK,), jnp.int32),
            pltpu.VMEM((CHUNK,), jnp.int32),
            pltpu.VMEM((CHUNK, D), jnp.float32),
            pltpu.VMEM((RCOPY, D), jnp.float32),
            pltpu.VMEM_SHARED((N, D), jnp.float32),
            pltpu.SemaphoreType.DMA,
        ],
    )
    part = f(ysplit, src3, dst3, ew)
    return jnp.concatenate([part[:N], part[N:]], axis=1)


def _graph_norm(x, alpha, gamma, beta):
    mean = jnp.mean(x, axis=0, keepdims=True)
    sub = x - alpha[None, :] * mean
    std = jnp.sqrt(jnp.mean(sub * sub, axis=0, keepdims=True) + EPS)
    return gamma[None, :] * sub / std + beta[None, :]


def _sort_pool(feat, k):
    fs = jnp.sort(feat, axis=-1)
    _, idx = jax.lax.top_k(fs[:, -1], k)
    sel = fs[idx]
    return sel.reshape(1, k * feat.shape[-1])


def kernel(features, edge_index, edge_weights, W1, W2, alpha1, gamma1, beta1, alpha2, gamma2, beta2):
    src = edge_index[0]
    dst = edge_index[1]
    src3 = src.reshape(NS, NCHUNK, CHUNK)
    dst3 = dst.reshape(NS, NCHUNK, CHUNK)

    out_deg = jnp.clip(jnp.bincount(src, length=N), 1, None).astype(jnp.float32)
    in_deg = jnp.clip(jnp.bincount(dst, length=N), 1, None).astype(jnp.float32)
    oscale = out_deg ** -0.5
    iscale = in_deg ** -0.5

    agg1 = _sc_edge_scatter(features * oscale[:, None], src3, dst3, edge_weights, 64)
    h1 = (agg1 * iscale[:, None]) @ W1
    h1 = _leaky(_graph_norm(h1, alpha1, gamma1, beta1))
    r1 = _sort_pool(h1, K)

    agg2 = _sc_edge_scatter(h1 * oscale[:, None], src3, dst3, edge_weights, 64)
    h2 = (agg2 * iscale[:, None]) @ W2
    h2 = _leaky(_graph_norm(h2, alpha2, gamma2, beta2))
    r2 = _sort_pool(h2, K)

    readouts = jnp.concatenate([r1, r2], axis=1)
    return pl.pallas_call(
        lambda x_ref, o_ref: o_ref.__setitem__(..., _leaky(x_ref[...])),
        out_shape=jax.ShapeDtypeStruct(readouts.shape, readouts.dtype),
    )(readouts)


# speedup vs baseline: 6.6425x; 1.8203x over previous
import functools

import jax
import jax.numpy as jnp
from jax import lax
from jax.experimental import pallas as pl
from jax.experimental.pallas import tpu as pltpu
from jax.experimental.pallas import tpu_sc as plsc

N = 10000
E = 320000
K = 64
EPS = 1e-05
NEG_SLOPE = 0.01

NC = 2
NS = 16
ET = E // NS
CHUNK = 80
NCHUNK = ET // CHUNK
RPW = 624
RCOPY = 104
RTAIL = N - NS * RPW


def _leaky(x):
    return jnp.where(x >= 0, x, NEG_SLOPE * x)


def _edge_scatter_body(D, y_hbm, src_hbm, dst_hbm, ew_hbm, out_hbm,
                       srcv, dstv, eww, idxb0, idxb1, dstb0, dstb1,
                       rows0, rows1, srows0, srows1, zbuf, acc,
                       sg0, sg1, ss0, ss1):
    idxb = (idxb0, idxb1)
    dstb = (dstb0, dstb1)
    rows = (rows0, rows1)
    srows = (srows0, srows1)
    sg = (sg0, sg1)
    ss = (ss0, ss1)
    cid = lax.axis_index("c")
    sid = lax.axis_index("s")

    def zero_row(r, _):
        for d in range(D // 16):
            zbuf[r, pl.ds(d * 16, 16)] = jnp.zeros((16,), jnp.float32)
        return 0

    lax.fori_loop(0, RCOPY, zero_row, 0)
    for j in range(RPW // RCOPY):
        pltpu.sync_copy(zbuf, acc.at[pl.ds(sid * RPW + j * RCOPY, RCOPY)])

    @pl.when(sid == NS - 1)
    def _():
        pltpu.sync_copy(zbuf.at[pl.ds(0, RTAIL)], acc.at[pl.ds(NS * RPW, RTAIL)])

    plsc.subcore_barrier()

    pltpu.sync_copy(src_hbm.at[sid], srcv)
    pltpu.sync_copy(dst_hbm.at[sid], dstv)
    pltpu.sync_copy(ew_hbm.at[pl.ds(sid * ET, ET)], eww)

    row_off = cid * N

    def fill_src(i, b):
        def adj(g, _):
            idxb[b][pl.ds(g * 16, 16)] = srcv[i, pl.ds(g * 16, 16)] + row_off
            return 0

        lax.fori_loop(0, CHUNK // 16, adj, 0)

    def fill_dst(i, b):
        def adj(g, _):
            dstb[b][pl.ds(g * 16, 16)] = dstv[i, pl.ds(g * 16, 16)]
            return 0

        lax.fori_loop(0, CHUNK // 16, adj, 0)

    for b in range(2):
        fill_src(b, b)
        pltpu.async_copy(y_hbm.at[idxb[b]], rows[b], sg[b])

    def step(s, _):
        for b in range(2):
            i = 2 * s + b
            pltpu.make_async_copy(y_hbm.at[idxb[b]], rows[b], sg[b]).wait()

            @pl.when(i >= 2)
            def _():
                pltpu.make_async_copy(srows[b], acc.at[dstb[b]], ss[b]).wait()

            def scale_group(g, _):
                ew16 = eww[pl.ds(i * CHUNK + g * 16, 16)]
                for j in range(16):
                    w = jnp.broadcast_to(ew16[j], (16,))
                    e = g * 16 + j
                    for d in range(D // 16):
                        srows[b][e, pl.ds(d * 16, 16)] = rows[b][e, pl.ds(d * 16, 16)] * w
                return 0

            lax.fori_loop(0, CHUNK // 16, scale_group, 0)

            @pl.when(i + 2 < NCHUNK)
            def _():
                fill_src(i + 2, b)
                pltpu.async_copy(y_hbm.at[idxb[b]], rows[b], sg[b])

            fill_dst(i, b)
            pltpu.async_copy(srows[b], acc.at[dstb[b]], ss[b], add=True)
        return 0

    lax.fori_loop(0, NCHUNK // 2, step, 0)
    for b in range(2):
        pltpu.make_async_copy(srows[b], acc.at[dstb[b]], ss[b]).wait()
    plsc.subcore_barrier()

    for j in range(RPW // RCOPY):
        r0 = sid * RPW + j * RCOPY
        pltpu.sync_copy(acc.at[pl.ds(r0, RCOPY)],
                        out_hbm.at[pl.ds(cid * N + r0, RCOPY)])

    @pl.when(sid == NS - 1)
    def _():
        pltpu.sync_copy(acc.at[pl.ds(NS * RPW, RTAIL)],
                        out_hbm.at[pl.ds(cid * N + NS * RPW, RTAIL)])


def _sc_edge_scatter(y, src3, dst3, ew, D):
    ysplit = jnp.concatenate([y[:, :D], y[:, D:]], axis=0)
    mesh = plsc.VectorSubcoreMesh(core_axis_name="c", subcore_axis_name="s")
    body = functools.partial(_edge_scatter_body, D)
    f = pl.kernel(
        body,
        out_type=jax.ShapeDtypeStruct((NC * N, D), jnp.float32),
        mesh=mesh,
        compiler_params=pltpu.CompilerParams(use_tc_tiling_on_sc=False),
        scratch_types=[
            pltpu.VMEM((NCHUNK, CHUNK), jnp.int32),
            pltpu.VMEM((NCHUNK, CHUNK), jnp.int32),
            pltpu.VMEM((ET,), jnp.float32),
            pltpu.VMEM((CHUNK,), jnp.int32),
            pltpu.VMEM((CHUNK,), jnp.int32),
            pltpu.VMEM((CHUNK,), jnp.int32),
            pltpu.VMEM((CHUNK,), jnp.int32),
            pltpu.VMEM((CHUNK, D), jnp.float32),
            pltpu.VMEM((CHUNK, D), jnp.float32),
            pltpu.VMEM((CHUNK, D), jnp.float32),
            pltpu.VMEM((CHUNK, D), jnp.float32),
            pltpu.VMEM((RCOPY, D), jnp.float32),
            pltpu.VMEM_SHARED((N, D), jnp.float32),
            pltpu.SemaphoreType.DMA,
            pltpu.SemaphoreType.DMA,
            pltpu.SemaphoreType.DMA,
            pltpu.SemaphoreType.DMA,
        ],
    )
    part = f(ysplit, src3, dst3, ew)
    return jnp.concatenate([part[:N], part[N:]], axis=1)


def _graph_norm(x, alpha, gamma, beta):
    mean = jnp.mean(x, axis=0, keepdims=True)
    sub = x - alpha[None, :] * mean
    std = jnp.sqrt(jnp.mean(sub * sub, axis=0, keepdims=True) + EPS)
    return gamma[None, :] * sub / std + beta[None, :]


def _sort_pool(feat, k):
    fs = jnp.sort(feat, axis=-1)
    _, idx = jax.lax.top_k(fs[:, -1], k)
    sel = fs[idx]
    return sel.reshape(1, k * feat.shape[-1])


def kernel(features, edge_index, edge_weights, W1, W2, alpha1, gamma1, beta1, alpha2, gamma2, beta2):
    src = edge_index[0]
    dst = edge_index[1]
    src3 = src.reshape(NS, NCHUNK, CHUNK)
    dst3 = dst.reshape(NS, NCHUNK, CHUNK)

    out_deg = jnp.clip(jnp.bincount(src, length=N), 1, None).astype(jnp.float32)
    in_deg = jnp.clip(jnp.bincount(dst, length=N), 1, None).astype(jnp.float32)
    oscale = out_deg ** -0.5
    iscale = in_deg ** -0.5

    agg1 = _sc_edge_scatter(features * oscale[:, None], src3, dst3, edge_weights, 64)
    h1 = (agg1 * iscale[:, None]) @ W1
    h1 = _leaky(_graph_norm(h1, alpha1, gamma1, beta1))
    r1 = _sort_pool(h1, K)

    agg2 = _sc_edge_scatter(h1 * oscale[:, None], src3, dst3, edge_weights, 64)
    h2 = (agg2 * iscale[:, None]) @ W2
    h2 = _leaky(_graph_norm(h2, alpha2, gamma2, beta2))
    r2 = _sort_pool(h2, K)

    readouts = jnp.concatenate([r1, r2], axis=1)
    return pl.pallas_call(
        lambda x_ref, o_ref: o_ref.__setitem__(..., _leaky(x_ref[...])),
        out_shape=jax.ShapeDtypeStruct(readouts.shape, readouts.dtype),
    )(readouts)


# speedup vs baseline: 7.4208x; 1.1172x over previous
import functools

import jax
import jax.numpy as jnp
from jax import lax
from jax.experimental import pallas as pl
from jax.experimental.pallas import tpu as pltpu
from jax.experimental.pallas import tpu_sc as plsc

N = 10000
E = 320000
K = 64
EPS = 1e-05
NEG_SLOPE = 0.01

NC = 2
NS = 16
ET = E // NS
CHUNK = 80
NCHUNK = ET // CHUNK
RPW = 624
RCOPY = 104
RTAIL = N - NS * RPW


def _leaky(x):
    return jnp.where(x >= 0, x, NEG_SLOPE * x)


def _edge_scatter_body(D, y_hbm, src_hbm, dst_hbm, ew_hbm, out_hbm,
                       srcv, dstv, eww, idxb0, idxb1, dstb0, dstb1,
                       rows0, rows1, srows0, srows1, zbuf, acc,
                       sg0, sg1, ss0, ss1):
    idxb = (idxb0, idxb1)
    dstb = (dstb0, dstb1)
    rows = (rows0, rows1)
    srows = (srows0, srows1)
    sg = (sg0, sg1)
    ss = (ss0, ss1)
    cid = lax.axis_index("c")
    sid = lax.axis_index("s")

    def zero_row(r, _):
        for d in range(D // 16):
            zbuf[r, pl.ds(d * 16, 16)] = jnp.zeros((16,), jnp.float32)
        return 0

    lax.fori_loop(0, RCOPY, zero_row, 0)
    for j in range(RPW // RCOPY):
        pltpu.sync_copy(zbuf, acc.at[pl.ds(sid * RPW + j * RCOPY, RCOPY)])

    @pl.when(sid == NS - 1)
    def _():
        pltpu.sync_copy(zbuf.at[pl.ds(0, RTAIL)], acc.at[pl.ds(NS * RPW, RTAIL)])

    plsc.subcore_barrier()

    pltpu.sync_copy(src_hbm.at[sid], srcv)
    pltpu.sync_copy(dst_hbm.at[sid], dstv)
    pltpu.sync_copy(ew_hbm.at[pl.ds(sid * ET, ET)], eww)

    row_off = cid * N

    def fill_src(i, b):
        def adj(g, _):
            idxb[b][pl.ds(g * 16, 16)] = srcv[i, pl.ds(g * 16, 16)] + row_off
            return 0

        lax.fori_loop(0, CHUNK // 16, adj, 0)

    def fill_dst(i, b):
        def adj(g, _):
            dstb[b][pl.ds(g * 16, 16)] = dstv[i, pl.ds(g * 16, 16)]
            return 0

        lax.fori_loop(0, CHUNK // 16, adj, 0)

    for b in range(2):
        fill_src(b, b)
        pltpu.async_copy(y_hbm.at[idxb[b]], rows[b], sg[b])

    def step(s, _):
        for b in range(2):
            i = 2 * s + b
            pltpu.make_async_copy(y_hbm.at[idxb[b]], rows[b], sg[b]).wait()

            @pl.when(i >= 2)
            def _():
                pltpu.make_async_copy(srows[b], acc.at[dstb[b]], ss[b]).wait()

            def scale_group(g, _):
                ew16 = eww[pl.ds(i * CHUNK + g * 16, 16)]
                for j in range(16):
                    w = jnp.broadcast_to(ew16[j], (16,))
                    e = g * 16 + j
                    for d in range(D // 16):
                        srows[b][e, pl.ds(d * 16, 16)] = rows[b][e, pl.ds(d * 16, 16)] * w
                return 0

            lax.fori_loop(0, CHUNK // 16, scale_group, 0)

            @pl.when(i + 2 < NCHUNK)
            def _():
                fill_src(i + 2, b)
                pltpu.async_copy(y_hbm.at[idxb[b]], rows[b], sg[b])

            fill_dst(i, b)
            pltpu.async_copy(srows[b], acc.at[dstb[b]], ss[b], add=True)
        return 0

    lax.fori_loop(0, NCHUNK // 2, step, 0)
    for b in range(2):
        pltpu.make_async_copy(srows[b], acc.at[dstb[b]], ss[b]).wait()
    plsc.subcore_barrier()

    for j in range(RPW // RCOPY):
        r0 = sid * RPW + j * RCOPY
        pltpu.sync_copy(acc.at[pl.ds(r0, RCOPY)],
                        out_hbm.at[pl.ds(cid * N + r0, RCOPY)])

    @pl.when(sid == NS - 1)
    def _():
        pltpu.sync_copy(acc.at[pl.ds(NS * RPW, RTAIL)],
                        out_hbm.at[pl.ds(cid * N + NS * RPW, RTAIL)])


def _sc_edge_scatter(y, src3, dst3, ew, D):
    ysplit = jnp.concatenate([y[:, :D], y[:, D:]], axis=0)
    mesh = plsc.VectorSubcoreMesh(core_axis_name="c", subcore_axis_name="s")
    body = functools.partial(_edge_scatter_body, D)
    f = pl.kernel(
        body,
        out_type=jax.ShapeDtypeStruct((NC * N, D), jnp.float32),
        mesh=mesh,
        compiler_params=pltpu.CompilerParams(use_tc_tiling_on_sc=False),
        scratch_types=[
            pltpu.VMEM((NCHUNK, CHUNK), jnp.int32),
            pltpu.VMEM((NCHUNK, CHUNK), jnp.int32),
            pltpu.VMEM((ET,), jnp.float32),
            pltpu.VMEM((CHUNK,), jnp.int32),
            pltpu.VMEM((CHUNK,), jnp.int32),
            pltpu.VMEM((CHUNK,), jnp.int32),
            pltpu.VMEM((CHUNK,), jnp.int32),
            pltpu.VMEM((CHUNK, D), jnp.float32),
            pltpu.VMEM((CHUNK, D), jnp.float32),
            pltpu.VMEM((CHUNK, D), jnp.float32),
            pltpu.VMEM((CHUNK, D), jnp.float32),
            pltpu.VMEM((RCOPY, D), jnp.float32),
            pltpu.VMEM_SHARED((N, D), jnp.float32),
            pltpu.SemaphoreType.DMA,
            pltpu.SemaphoreType.DMA,
            pltpu.SemaphoreType.DMA,
            pltpu.SemaphoreType.DMA,
        ],
    )
    part = f(ysplit, src3, dst3, ew)
    return jnp.concatenate([part[:N], part[N:]], axis=1)


def _graph_norm(x, alpha, gamma, beta):
    mean = jnp.mean(x, axis=0, keepdims=True)
    sub = x - alpha[None, :] * mean
    std = jnp.sqrt(jnp.mean(sub * sub, axis=0, keepdims=True) + EPS)
    return gamma[None, :] * sub / std + beta[None, :]


def _sort_pool(feat, k):
    key = jnp.max(feat, axis=-1)
    _, idx = jax.lax.top_k(key, k)
    sel = jnp.sort(feat[idx], axis=-1)
    return sel.reshape(1, k * feat.shape[-1])


def kernel(features, edge_index, edge_weights, W1, W2, alpha1, gamma1, beta1, alpha2, gamma2, beta2):
    src = edge_index[0]
    dst = edge_index[1]
    src3 = src.reshape(NS, NCHUNK, CHUNK)
    dst3 = dst.reshape(NS, NCHUNK, CHUNK)

    out_deg = jnp.clip(jnp.bincount(src, length=N), 1, None).astype(jnp.float32)
    in_deg = jnp.clip(jnp.bincount(dst, length=N), 1, None).astype(jnp.float32)
    oscale = out_deg ** -0.5
    iscale = in_deg ** -0.5

    agg1 = _sc_edge_scatter(features * oscale[:, None], src3, dst3, edge_weights, 64)
    h1 = (agg1 * iscale[:, None]) @ W1
    h1 = _leaky(_graph_norm(h1, alpha1, gamma1, beta1))
    r1 = _sort_pool(h1, K)

    agg2 = _sc_edge_scatter(h1 * oscale[:, None], src3, dst3, edge_weights, 64)
    h2 = (agg2 * iscale[:, None]) @ W2
    h2 = _leaky(_graph_norm(h2, alpha2, gamma2, beta2))
    r2 = _sort_pool(h2, K)

    readouts = jnp.concatenate([r1, r2], axis=1)
    return pl.pallas_call(
        lambda x_ref, o_ref: o_ref.__setitem__(..., _leaky(x_ref[...])),
        out_shape=jax.ShapeDtypeStruct(readouts.shape, readouts.dtype),
    )(readouts)
